# baseline (speedup 1.0000x reference)
import functools

import jax
import jax.numpy as jnp
from jax.experimental import pallas as pl
from jax.experimental.pallas import tpu as pltpu

N_EXPERTS = 8
N_GROUP = 4
GROUP_SZ = N_EXPERTS // N_GROUP
ROUTED_SCALE = 2.5
NEG_INF = float("-inf")


def _relu2(x):
    r = jnp.maximum(x, 0.0)
    return r * r


def _route_body(logits_t_ref, bias_ref, comb_ref):
    lt = logits_t_ref[...]
    s = 1.0 / (1.0 + jnp.exp(-lt))
    sb = s + bias_ref[...]
    g = [sb[2 * i:2 * i + 1, :] + sb[2 * i + 1:2 * i + 2, :]
         for i in range(N_GROUP)]
    gsel = []
    for i in range(N_GROUP):
        rank = jnp.zeros_like(g[i], dtype=jnp.int32)
        for j in range(N_GROUP):
            if j == i:
                continue
            gt = g[j] > g[i]
            tie = (g[j] == g[i]) & (j < i)
            rank = rank + (gt | tie).astype(jnp.int32)
        gsel.append(rank < 2)
    ms = [jnp.where(gsel[e // GROUP_SZ], sb[e:e + 1, :], NEG_INF)
          for e in range(N_EXPERTS)]
    rows = []
    for i in range(N_EXPERTS):
        rank = jnp.zeros_like(ms[i], dtype=jnp.int32)
        for j in range(N_EXPERTS):
            if j == i:
                continue
            gt = ms[j] > ms[i]
            tie = (ms[j] == ms[i]) & (j < i)
            rank = rank + (gt | tie).astype(jnp.int32)
        rows.append((rank < 2).astype(jnp.float32) * s[i:i + 1, :])
    w = jnp.concatenate(rows, axis=0)
    denom = jnp.sum(w, axis=0, keepdims=True) + 1e-20
    wt = w * (ROUTED_SCALE / denom)
    comb_ref[...] = jax.lax.dot_general(
        wt, jnp.eye(N_EXPERTS, dtype=jnp.float32),
        (((0,), (0,)), ((), ())), preferred_element_type=jnp.float32)


def _moe_body(x_ref, comb_ref, w1_ref, w2_ref, ws1_ref, ws2_ref,
              out_ref):
    j = pl.program_id(1)

    @pl.when(j == 0)
    def _shared():
        x = x_ref[...]
        h = _relu2(jnp.dot(x, ws1_ref[...],
                           preferred_element_type=jnp.float32))
        out_ref[...] = jnp.dot(h, ws2_ref[...],
                               preferred_element_type=jnp.float32)

    @pl.when(j > 0)
    def _expert():
        lane = jax.lax.broadcasted_iota(jnp.int32, comb_ref.shape, 1)
        acc = None
        for k in range(2):
            e = 2 * (j - 1) + k
            h = _relu2(jnp.dot(x_ref[...], w1_ref[k],
                               preferred_element_type=jnp.float32))
            y = jnp.dot(h, w2_ref[k],
                        preferred_element_type=jnp.float32)
            ce = jnp.sum(jnp.where(lane == e, comb_ref[...], 0.0),
                         axis=1, keepdims=True)
            acc = ce * y if acc is None else acc + ce * y
        out_ref[...] += acc


@functools.partial(jax.jit, static_argnames=("tm",))
def _moe_fused(hidden_states, logits, gate_bias, w1, w2, ws1, ws2, tm=1024):
    T, D = hidden_states.shape
    E, _, F = w1.shape
    SF = ws1.shape[1]
    comb = pl.pallas_call(
        _route_body,
        in_specs=[pl.BlockSpec((N_EXPERTS, T), lambda: (0, 0)),
                  pl.BlockSpec((N_EXPERTS, 1), lambda: (0, 0))],
        out_specs=pl.BlockSpec((T, N_EXPERTS), lambda: (0, 0)),
        out_shape=jax.ShapeDtypeStruct((T, N_EXPERTS), jnp.float32),
    )(logits.T, gate_bias.reshape(N_EXPERTS, 1))
    grid = (T // tm, 1 + E // 2)
    return pl.pallas_call(
        _moe_body,
        grid=grid,
        in_specs=[
            pl.BlockSpec((tm, D), lambda m, j: (m, 0)),
            pl.BlockSpec((tm, N_EXPERTS), lambda m, j: (m, 0)),
            pl.BlockSpec((2, D, F), lambda m, j: (jnp.maximum(j, 1) - 1, 0, 0)),
            pl.BlockSpec((2, F, D), lambda m, j: (jnp.maximum(j, 1) - 1, 0, 0)),
            pl.BlockSpec((D, SF), lambda m, j: (0, 0)),
            pl.BlockSpec((SF, D), lambda m, j: (0, 0)),
        ],
        out_specs=pl.BlockSpec((tm, D), lambda m, j: (m, 0)),
        out_shape=jax.ShapeDtypeStruct((T, D), jnp.float32),
        compiler_params=pltpu.CompilerParams(
            dimension_semantics=("parallel", "arbitrary")),
    )(hidden_states, comb, w1, w2, ws1, ws2)


def kernel(hidden_states, gate_w, gate_bias, w1, w2, ws1, ws2):
    logits = jnp.dot(hidden_states.astype(jnp.float32), gate_w.T)
    return _moe_fused(hidden_states, logits, gate_bias, w1, w2, ws1, ws2)
